# Initial kernel scaffold; baseline (speedup 1.0000x reference)
import jax
import jax.numpy as jnp
from jax.experimental import pallas as pl


def kernel(coords, edge_index):
    raise NotImplementedError("write your pallas kernel here")


# speedup vs baseline: 6.0872x; 6.0872x over previous
import functools

import jax
import jax.numpy as jnp
from jax import lax
from jax.experimental import pallas as pl
from jax.experimental.pallas import tpu as pltpu
from jax.experimental.pallas import tpu_sc as plsc

N = 10000
E = 160000
R = 160
C = 1024
EP = R * C
BOND_TARGET = 1.5
ANGLE_TARGET = 1.9111355309200186
PI = 3.141592653589793

_NC, _NS, _L = 2, 16, 16
NW = _NC * _NS
CH = EP // NW


def _sc_body(cx, cy, cz, srcs, dsts, ox, oy, oz,
             cxv, cyv, czv, sv, dv, dxv, dyv, dzv):
    wid = lax.axis_index("s") * _NC + lax.axis_index("c")
    base = wid * CH
    pltpu.sync_copy(cx.at[:], cxv)
    pltpu.sync_copy(cy.at[:], cyv)
    pltpu.sync_copy(cz.at[:], czv)
    pltpu.sync_copy(srcs.at[pl.ds(base, CH)], sv)
    pltpu.sync_copy(dsts.at[pl.ds(base, CH)], dv)

    def step(i, _):
        off = i * _L
        s16 = sv[pl.ds(off, _L)]
        d16 = dv[pl.ds(off, _L)]
        dxv[pl.ds(off, _L)] = (plsc.load_gather(cxv, [d16])
                               - plsc.load_gather(cxv, [s16]))
        dyv[pl.ds(off, _L)] = (plsc.load_gather(cyv, [d16])
                               - plsc.load_gather(cyv, [s16]))
        dzv[pl.ds(off, _L)] = (plsc.load_gather(czv, [d16])
                               - plsc.load_gather(czv, [s16]))
        return 0

    lax.fori_loop(0, CH // _L, step, 0)
    pltpu.sync_copy(dxv, ox.at[pl.ds(base, CH)])
    pltpu.sync_copy(dyv, oy.at[pl.ds(base, CH)])
    pltpu.sync_copy(dzv, oz.at[pl.ds(base, CH)])


@functools.cache
def _sc_gather():
    return pl.kernel(
        _sc_body,
        out_type=[jax.ShapeDtypeStruct((EP,), jnp.float32)] * 3,
        mesh=plsc.VectorSubcoreMesh(core_axis_name="c", subcore_axis_name="s"),
        compiler_params=pltpu.CompilerParams(needs_layout_passes=False),
        scratch_types=(
            [pltpu.VMEM((N,), jnp.float32)] * 3
            + [pltpu.VMEM((CH,), jnp.int32)] * 2
            + [pltpu.VMEM((CH,), jnp.float32)] * 3
        ),
    )


def _acos(x):
    a = jnp.abs(x)
    p = jnp.float32(-0.0012624911)
    p = p * a + jnp.float32(0.0066700901)
    p = p * a + jnp.float32(-0.0170881256)
    p = p * a + jnp.float32(0.0308918810)
    p = p * a + jnp.float32(-0.0501743046)
    p = p * a + jnp.float32(0.0889789874)
    p = p * a + jnp.float32(-0.2145988016)
    p = p * a + jnp.float32(1.5707963050)
    r = jnp.sqrt(jnp.float32(1.0) - a) * p
    return jnp.where(x < 0, jnp.float32(PI) - r, r)


def _flat_roll1(a):
    left = a[:, 1:]
    wrap = jnp.roll(a[:, :1], -1, axis=0)
    return jnp.concatenate([left, wrap], axis=1)


def _tc_body(src_ref, dx_ref, dy_ref, dz_ref, out_ref):
    src = src_ref[...]
    dx = dx_ref[...]
    dy = dy_ref[...]
    dz = dz_ref[...]
    idx = (lax.broadcasted_iota(jnp.int32, (R, C), 0) * C
           + lax.broadcasted_iota(jnp.int32, (R, C), 1))

    dist2 = dx * dx + dy * dy + dz * dz
    dist = jnp.sqrt(dist2 + jnp.float32(1e-8))
    bdev = dist - jnp.float32(BOND_TARGET)
    bond_sum = jnp.sum(jnp.where(idx < E, bdev * bdev, jnp.float32(0.0)))

    inv = jnp.float32(1.0) / (jnp.sqrt(dist2) + jnp.float32(1e-8))
    ux = dx * inv
    uy = dy * inv
    uz = dz * inv

    def cond(carry):
        return carry[1] > 0

    def body(carry):
        d, _, ssh, xsh, ysh, zsh, sq_sum, cnt = carry
        ssh = _flat_roll1(ssh)
        xsh = _flat_roll1(xsh)
        ysh = _flat_roll1(ysh)
        zsh = _flat_roll1(zsh)
        valid = (idx < E - d) & (src == ssh)
        cos = jnp.clip(ux * xsh + uy * ysh + uz * zsh,
                       jnp.float32(-0.999), jnp.float32(0.999))
        dev = _acos(cos) - jnp.float32(ANGLE_TARGET)
        sq_sum = sq_sum + jnp.sum(jnp.where(valid, dev * dev,
                                            jnp.float32(0.0)))
        nv = jnp.sum(valid.astype(jnp.int32))
        return (d + 1, nv, ssh, xsh, ysh, zsh, sq_sum, cnt + nv)

    carry0 = (jnp.int32(1), jnp.int32(1), src, ux, uy, uz,
              jnp.float32(0.0), jnp.int32(0))
    res = lax.while_loop(cond, body, carry0)
    sq_sum = res[6]
    cnt = res[7]
    loss = (bond_sum / jnp.float32(E)
            + sq_sum / jnp.maximum(cnt, 1).astype(jnp.float32))
    out_ref[...] = jnp.broadcast_to(loss, (1, 1))


_tc_loss = pl.pallas_call(
    _tc_body,
    out_shape=jax.ShapeDtypeStruct((1, 1), jnp.float32),
)


def kernel(coords, edge_index):
    src = edge_index[0]
    dst = edge_index[1]
    order = jnp.argsort(src)
    pad = jnp.zeros((EP - E,), jnp.int32)
    src_s = jnp.concatenate([src[order], pad])
    dst_s = jnp.concatenate([dst[order], pad])
    cx = coords[:, 0]
    cy = coords[:, 1]
    cz = coords[:, 2]
    dx, dy, dz = _sc_gather()(cx, cy, cz, src_s, dst_s)
    out = _tc_loss(src_s.reshape(R, C), dx.reshape(R, C),
                   dy.reshape(R, C), dz.reshape(R, C))
    return out[0, 0]


# speedup vs baseline: 6.7183x; 1.1037x over previous
import functools

import jax
import jax.numpy as jnp
from jax import lax
from jax.experimental import pallas as pl
from jax.experimental.pallas import tpu as pltpu
from jax.experimental.pallas import tpu_sc as plsc

N = 10000
E = 160000
R = 160
C = 1024
EP = R * C
BOND_TARGET = 1.5
ANGLE_TARGET = 1.9111355309200186
PI = 3.141592653589793

_NC, _NS, _L = 2, 16, 16
NW = _NC * _NS
CH = EP // NW


def _sc_body(cx, cy, cz, srcs, dsts, ox, oy, oz,
             cxv, cyv, czv, sv, dv, dxv, dyv, dzv):
    wid = lax.axis_index("s") * _NC + lax.axis_index("c")
    base = wid * CH
    pltpu.sync_copy(cx.at[:], cxv)
    pltpu.sync_copy(cy.at[:], cyv)
    pltpu.sync_copy(cz.at[:], czv)
    pltpu.sync_copy(srcs.at[pl.ds(base, CH)], sv)
    pltpu.sync_copy(dsts.at[pl.ds(base, CH)], dv)

    def step(i, _):
        off = i * _L
        s16 = sv[pl.ds(off, _L)]
        d16 = dv[pl.ds(off, _L)]
        dxv[pl.ds(off, _L)] = (plsc.load_gather(cxv, [d16])
                               - plsc.load_gather(cxv, [s16]))
        dyv[pl.ds(off, _L)] = (plsc.load_gather(cyv, [d16])
                               - plsc.load_gather(cyv, [s16]))
        dzv[pl.ds(off, _L)] = (plsc.load_gather(czv, [d16])
                               - plsc.load_gather(czv, [s16]))
        return 0

    lax.fori_loop(0, CH // _L, step, 0)
    pltpu.sync_copy(dxv, ox.at[pl.ds(base, CH)])
    pltpu.sync_copy(dyv, oy.at[pl.ds(base, CH)])
    pltpu.sync_copy(dzv, oz.at[pl.ds(base, CH)])


@functools.cache
def _sc_gather():
    return pl.kernel(
        _sc_body,
        out_type=[jax.ShapeDtypeStruct((EP,), jnp.float32)] * 3,
        mesh=plsc.VectorSubcoreMesh(core_axis_name="c", subcore_axis_name="s"),
        compiler_params=pltpu.CompilerParams(needs_layout_passes=False),
        scratch_types=(
            [pltpu.VMEM((N,), jnp.float32)] * 3
            + [pltpu.VMEM((CH,), jnp.int32)] * 2
            + [pltpu.VMEM((CH,), jnp.float32)] * 3
        ),
    )


def _acos(x):
    a = jnp.abs(x)
    p = jnp.float32(-0.0012624911)
    p = p * a + jnp.float32(0.0066700901)
    p = p * a + jnp.float32(-0.0170881256)
    p = p * a + jnp.float32(0.0308918810)
    p = p * a + jnp.float32(-0.0501743046)
    p = p * a + jnp.float32(0.0889789874)
    p = p * a + jnp.float32(-0.2145988016)
    p = p * a + jnp.float32(1.5707963050)
    r = jnp.sqrt(jnp.float32(1.0) - a) * p
    return jnp.where(x < 0, jnp.float32(PI) - r, r)


def _flat_roll1(a):
    left = a[:, 1:]
    wrap = jnp.roll(a[:, :1], -1, axis=0)
    return jnp.concatenate([left, wrap], axis=1)


def _tc_body(src_ref, dx_ref, dy_ref, dz_ref, out_ref):
    src = src_ref[...]
    dx = dx_ref[...]
    dy = dy_ref[...]
    dz = dz_ref[...]
    idx = (lax.broadcasted_iota(jnp.int32, (R, C), 0) * C
           + lax.broadcasted_iota(jnp.int32, (R, C), 1))

    dist2 = dx * dx + dy * dy + dz * dz
    dist = jnp.sqrt(dist2 + jnp.float32(1e-8))
    bdev = dist - jnp.float32(BOND_TARGET)
    bond_sum = jnp.sum(jnp.where(idx < E, bdev * bdev, jnp.float32(0.0)))

    inv = jnp.float32(1.0) / (jnp.sqrt(dist2) + jnp.float32(1e-8))
    ux = dx * inv
    uy = dy * inv
    uz = dz * inv

    def cond(carry):
        return carry[1] > 0

    def body(carry):
        d, _, ssh, xsh, ysh, zsh, sq_sum, cnt = carry
        ssh = _flat_roll1(ssh)
        xsh = _flat_roll1(xsh)
        ysh = _flat_roll1(ysh)
        zsh = _flat_roll1(zsh)
        valid = src == ssh
        cos = jnp.clip(ux * xsh + uy * ysh + uz * zsh,
                       jnp.float32(-0.999), jnp.float32(0.999))
        dev = _acos(cos) - jnp.float32(ANGLE_TARGET)
        sq_sum = sq_sum + jnp.sum(jnp.where(valid, dev * dev,
                                            jnp.float32(0.0)))
        nv = jnp.sum(valid.astype(jnp.int32))
        return (d + 1, nv, ssh, xsh, ysh, zsh, sq_sum, cnt + nv)

    carry0 = (jnp.int32(1), jnp.int32(1), src, ux, uy, uz,
              jnp.float32(0.0), jnp.int32(0))
    res = lax.while_loop(cond, body, carry0)
    sq_sum = res[6]
    cnt = res[7]
    loss = (bond_sum / jnp.float32(E)
            + sq_sum / jnp.maximum(cnt, 1).astype(jnp.float32))
    out_ref[...] = jnp.broadcast_to(loss, (1, 1))


_tc_loss = pl.pallas_call(
    _tc_body,
    out_shape=jax.ShapeDtypeStruct((1, 1), jnp.float32),
)


def kernel(coords, edge_index):
    src = edge_index[0]
    dst = edge_index[1]
    key = jnp.sort(src * 16384 + dst)
    src_s = key >> 14
    dst_s = key & 16383
    zpad = jnp.zeros((EP - E,), jnp.int32)
    negpad = -1 - jnp.arange(EP - E, dtype=jnp.int32)
    src_tc = jnp.concatenate([src_s, negpad])
    src_sc = jnp.concatenate([src_s, zpad])
    dst_sc = jnp.concatenate([dst_s, zpad])
    cx = coords[:, 0]
    cy = coords[:, 1]
    cz = coords[:, 2]
    dx, dy, dz = _sc_gather()(cx, cy, cz, src_sc, dst_sc)
    out = _tc_loss(src_tc.reshape(R, C), dx.reshape(R, C),
                   dy.reshape(R, C), dz.reshape(R, C))
    return out[0, 0]


# speedup vs baseline: 6.8298x; 1.0166x over previous
import functools

import jax
import jax.numpy as jnp
from jax import lax
from jax.experimental import pallas as pl
from jax.experimental.pallas import tpu as pltpu
from jax.experimental.pallas import tpu_sc as plsc

N = 10000
E = 160000
R = 160
C = 1024
EP = R * C
BOND_TARGET = 1.5
ANGLE_TARGET = 1.9111355309200186
PI = 3.141592653589793

_NC, _NS, _L = 2, 16, 16
NW = _NC * _NS
CH = EP // NW


def _sc_body(cx, cy, cz, srcs, dsts, ox, oy, oz,
             cxv, cyv, czv, sv, dv, dxv, dyv, dzv):
    wid = lax.axis_index("s") * _NC + lax.axis_index("c")
    base = wid * CH
    pltpu.sync_copy(cx.at[:], cxv)
    pltpu.sync_copy(cy.at[:], cyv)
    pltpu.sync_copy(cz.at[:], czv)
    pltpu.sync_copy(srcs.at[pl.ds(base, CH)], sv)
    pltpu.sync_copy(dsts.at[pl.ds(base, CH)], dv)

    def step(i, _):
        off = i * _L
        s16 = sv[pl.ds(off, _L)]
        d16 = dv[pl.ds(off, _L)]
        dxv[pl.ds(off, _L)] = (plsc.load_gather(cxv, [d16])
                               - plsc.load_gather(cxv, [s16]))
        dyv[pl.ds(off, _L)] = (plsc.load_gather(cyv, [d16])
                               - plsc.load_gather(cyv, [s16]))
        dzv[pl.ds(off, _L)] = (plsc.load_gather(czv, [d16])
                               - plsc.load_gather(czv, [s16]))
        return 0

    lax.fori_loop(0, CH // _L, step, 0)
    pltpu.sync_copy(dxv, ox.at[pl.ds(base, CH)])
    pltpu.sync_copy(dyv, oy.at[pl.ds(base, CH)])
    pltpu.sync_copy(dzv, oz.at[pl.ds(base, CH)])


@functools.cache
def _sc_gather():
    return pl.kernel(
        _sc_body,
        out_type=[jax.ShapeDtypeStruct((EP,), jnp.float32)] * 3,
        mesh=plsc.VectorSubcoreMesh(core_axis_name="c", subcore_axis_name="s"),
        compiler_params=pltpu.CompilerParams(needs_layout_passes=False),
        scratch_types=(
            [pltpu.VMEM((N,), jnp.float32)] * 3
            + [pltpu.VMEM((CH,), jnp.int32)] * 2
            + [pltpu.VMEM((CH,), jnp.float32)] * 3
        ),
    )


def _acos(x):
    a = jnp.abs(x)
    p = jnp.float32(-0.0187293)
    p = p * a + jnp.float32(0.0742610)
    p = p * a + jnp.float32(-0.2121144)
    p = p * a + jnp.float32(1.5707288)
    r = jnp.sqrt(jnp.float32(1.0) - a) * p
    return jnp.where(x < 0, jnp.float32(PI) - r, r)


def _flat_roll1(a):
    left = a[:, 1:]
    wrap = jnp.roll(a[:, :1], -1, axis=0)
    return jnp.concatenate([left, wrap], axis=1)


def _tc_body(src_ref, dx_ref, dy_ref, dz_ref, out_ref):
    src = src_ref[...]
    dx = dx_ref[...]
    dy = dy_ref[...]
    dz = dz_ref[...]
    idx = (lax.broadcasted_iota(jnp.int32, (R, C), 0) * C
           + lax.broadcasted_iota(jnp.int32, (R, C), 1))

    dist2 = dx * dx + dy * dy + dz * dz
    dist = jnp.sqrt(dist2 + jnp.float32(1e-8))
    bdev = dist - jnp.float32(BOND_TARGET)
    bond_sum = jnp.sum(jnp.where(idx < E, bdev * bdev, jnp.float32(0.0)))

    inv = jnp.float32(1.0) / (jnp.sqrt(dist2) + jnp.float32(1e-8))
    ux = dx * inv
    uy = dy * inv
    uz = dz * inv

    def cond(carry):
        return carry[1] > 0

    def _one_offset(ssh, xsh, ysh, zsh, sq_sum):
        valid = src == ssh
        cos = jnp.clip(ux * xsh + uy * ysh + uz * zsh,
                       jnp.float32(-0.999), jnp.float32(0.999))
        dev = _acos(cos) - jnp.float32(ANGLE_TARGET)
        sq_sum = sq_sum + jnp.sum(jnp.where(valid, dev * dev,
                                            jnp.float32(0.0)))
        return sq_sum, jnp.sum(valid.astype(jnp.int32))

    def body(carry):
        d, _, ssh, xsh, ysh, zsh, sq_sum, cnt = carry
        ssh = _flat_roll1(ssh)
        xsh = _flat_roll1(xsh)
        ysh = _flat_roll1(ysh)
        zsh = _flat_roll1(zsh)
        sq_sum, n1 = _one_offset(ssh, xsh, ysh, zsh, sq_sum)
        ssh = _flat_roll1(ssh)
        xsh = _flat_roll1(xsh)
        ysh = _flat_roll1(ysh)
        zsh = _flat_roll1(zsh)
        sq_sum, n2 = _one_offset(ssh, xsh, ysh, zsh, sq_sum)
        return (d + 2, n1 + n2, ssh, xsh, ysh, zsh, sq_sum, cnt + n1 + n2)

    carry0 = (jnp.int32(1), jnp.int32(1), src, ux, uy, uz,
              jnp.float32(0.0), jnp.int32(0))
    res = lax.while_loop(cond, body, carry0)
    sq_sum = res[6]
    cnt = res[7]
    loss = (bond_sum / jnp.float32(E)
            + sq_sum / jnp.maximum(cnt, 1).astype(jnp.float32))
    out_ref[...] = jnp.broadcast_to(loss, (1, 1))


_tc_loss = pl.pallas_call(
    _tc_body,
    out_shape=jax.ShapeDtypeStruct((1, 1), jnp.float32),
)


def kernel(coords, edge_index):
    src = edge_index[0]
    dst = edge_index[1]
    key = jnp.sort(src * 16384 + dst)
    src_s = key >> 14
    dst_s = key & 16383
    zpad = jnp.zeros((EP - E,), jnp.int32)
    negpad = -1 - jnp.arange(EP - E, dtype=jnp.int32)
    src_tc = jnp.concatenate([src_s, negpad])
    src_sc = jnp.concatenate([src_s, zpad])
    dst_sc = jnp.concatenate([dst_s, zpad])
    cx = coords[:, 0]
    cy = coords[:, 1]
    cz = coords[:, 2]
    dx, dy, dz = _sc_gather()(cx, cy, cz, src_sc, dst_sc)
    out = _tc_loss(src_tc.reshape(R, C), dx.reshape(R, C),
                   dy.reshape(R, C), dz.reshape(R, C))
    return out[0, 0]


# speedup vs baseline: 12.2683x; 1.7963x over previous
import functools

import jax
import jax.numpy as jnp
from jax import lax
from jax.experimental import pallas as pl
from jax.experimental.pallas import tpu as pltpu
from jax.experimental.pallas import tpu_sc as plsc

N = 10000
E = 160000
R = 160
C = 1024
EP = R * C
BOND_TARGET = 1.5
ANGLE_TARGET = 1.9111355309200186
PI = 3.141592653589793

_NC, _NS, _L = 2, 16, 16
NW = _NC * _NS
CH = EP // NW


def _sc_body(cx, cy, cz, srcs, dsts, ox, oy, oz,
             cxv, cyv, czv, sv, dv, dxv, dyv, dzv):
    wid = lax.axis_index("s") * _NC + lax.axis_index("c")
    base = wid * CH
    pltpu.sync_copy(cx.at[:], cxv)
    pltpu.sync_copy(cy.at[:], cyv)
    pltpu.sync_copy(cz.at[:], czv)
    pltpu.sync_copy(srcs.at[pl.ds(base, CH)], sv)
    pltpu.sync_copy(dsts.at[pl.ds(base, CH)], dv)

    def step(i, _):
        off = i * _L
        s16 = sv[pl.ds(off, _L)]
        d16 = dv[pl.ds(off, _L)]
        dxv[pl.ds(off, _L)] = (plsc.load_gather(cxv, [d16])
                               - plsc.load_gather(cxv, [s16]))
        dyv[pl.ds(off, _L)] = (plsc.load_gather(cyv, [d16])
                               - plsc.load_gather(cyv, [s16]))
        dzv[pl.ds(off, _L)] = (plsc.load_gather(czv, [d16])
                               - plsc.load_gather(czv, [s16]))
        return 0

    lax.fori_loop(0, CH // _L, step, 0)
    pltpu.sync_copy(dxv, ox.at[pl.ds(base, CH)])
    pltpu.sync_copy(dyv, oy.at[pl.ds(base, CH)])
    pltpu.sync_copy(dzv, oz.at[pl.ds(base, CH)])


@functools.cache
def _sc_gather():
    return pl.kernel(
        _sc_body,
        out_type=[jax.ShapeDtypeStruct((EP,), jnp.float32)] * 3,
        mesh=plsc.VectorSubcoreMesh(core_axis_name="c", subcore_axis_name="s"),
        compiler_params=pltpu.CompilerParams(needs_layout_passes=False),
        scratch_types=(
            [pltpu.VMEM((N,), jnp.float32)] * 3
            + [pltpu.VMEM((CH,), jnp.int32)] * 2
            + [pltpu.VMEM((CH,), jnp.float32)] * 3
        ),
    )


def _acos(x):
    a = jnp.abs(x)
    p = jnp.float32(-0.0187293)
    p = p * a + jnp.float32(0.0742610)
    p = p * a + jnp.float32(-0.2121144)
    p = p * a + jnp.float32(1.5707288)
    r = jnp.sqrt(jnp.float32(1.0) - a) * p
    return jnp.where(x < 0, jnp.float32(PI) - r, r)


def _flat_roll1(a):
    left = a[:, 1:]
    wrap = jnp.roll(a[:, :1], -1, axis=0)
    return jnp.concatenate([left, wrap], axis=1)


def _tc_body(src_ref, dx_ref, dy_ref, dz_ref, out_ref):
    src = src_ref[...]
    dx = dx_ref[...]
    dy = dy_ref[...]
    dz = dz_ref[...]
    idx = (lax.broadcasted_iota(jnp.int32, (R, C), 0) * C
           + lax.broadcasted_iota(jnp.int32, (R, C), 1))

    dist2 = dx * dx + dy * dy + dz * dz
    dist = jnp.sqrt(dist2 + jnp.float32(1e-8))
    bdev = dist - jnp.float32(BOND_TARGET)
    bond_sum = jnp.sum(jnp.where(idx < E, bdev * bdev, jnp.float32(0.0)))

    inv = jnp.float32(1.0) / (jnp.sqrt(dist2) + jnp.float32(1e-8))
    ux = dx * inv
    uy = dy * inv
    uz = dz * inv

    def cond(carry):
        return carry[1] > 0

    def _one_offset(ssh, xsh, ysh, zsh, sq_sum):
        valid = src == ssh
        cos = jnp.clip(ux * xsh + uy * ysh + uz * zsh,
                       jnp.float32(-0.999), jnp.float32(0.999))
        dev = _acos(cos) - jnp.float32(ANGLE_TARGET)
        sq_sum = sq_sum + jnp.sum(jnp.where(valid, dev * dev,
                                            jnp.float32(0.0)))
        return sq_sum, jnp.sum(valid.astype(jnp.int32))

    def body(carry):
        d, _, ssh, xsh, ysh, zsh, sq_sum, cnt = carry
        ssh = _flat_roll1(ssh)
        xsh = _flat_roll1(xsh)
        ysh = _flat_roll1(ysh)
        zsh = _flat_roll1(zsh)
        sq_sum, n1 = _one_offset(ssh, xsh, ysh, zsh, sq_sum)
        ssh = _flat_roll1(ssh)
        xsh = _flat_roll1(xsh)
        ysh = _flat_roll1(ysh)
        zsh = _flat_roll1(zsh)
        sq_sum, n2 = _one_offset(ssh, xsh, ysh, zsh, sq_sum)
        return (d + 2, n1 + n2, ssh, xsh, ysh, zsh, sq_sum, cnt + n1 + n2)

    carry0 = (jnp.int32(1), jnp.int32(1), src, ux, uy, uz,
              jnp.float32(0.0), jnp.int32(0))
    res = lax.while_loop(cond, body, carry0)
    sq_sum = res[6]
    cnt = res[7]
    loss = (bond_sum / jnp.float32(E)
            + sq_sum / jnp.maximum(cnt, 1).astype(jnp.float32))
    out_ref[...] = jnp.broadcast_to(loss, (1, 1))


_tc_loss = pl.pallas_call(
    _tc_body,
    out_shape=jax.ShapeDtypeStruct((1, 1), jnp.float32),
)


def kernel(coords, edge_index):
    src = edge_index[0]
    dst = edge_index[1]
    key = jnp.sort(src * 16384 + dst, stable=False)
    src_s = key >> 14
    dst_s = key & 16383
    zpad = jnp.zeros((EP - E,), jnp.int32)
    negpad = -1 - jnp.arange(EP - E, dtype=jnp.int32)
    src_tc = jnp.concatenate([src_s, negpad])
    src_sc = jnp.concatenate([src_s, zpad])
    dst_sc = jnp.concatenate([dst_s, zpad])
    cx = coords[:, 0]
    cy = coords[:, 1]
    cz = coords[:, 2]
    dx, dy, dz = _sc_gather()(cx, cy, cz, src_sc, dst_sc)
    out = _tc_loss(src_tc.reshape(R, C), dx.reshape(R, C),
                   dy.reshape(R, C), dz.reshape(R, C))
    return out[0, 0]
